# baseline (speedup 1.0000x reference)
import functools

import jax
import jax.numpy as jnp
from jax import lax
from jax.experimental import pallas as pl
from jax.experimental.pallas import tpu as pltpu
from jax.experimental.pallas import tpu_sc as plsc

_MOMENTUM = 0.01

_CSC = 99840
_NC = 2
_NS = 16
_NW = _NC * _NS
_CH = 8
_NBUF = 3

_RC = 2048


def _tc_body(x_ref, tgt_ref, pen_ref, o_ref):
    t = (1.0 - _MOMENTUM) * jnp.mean(tgt_ref[...])
    x = x_ref[...]
    p = pen_ref[...]
    o_ref[...] = jnp.where(x > p, x * (t + x), x)


def _tc_part(xt_top, tgt, pen):
    Ctc, B = xt_top.shape
    grid = (pl.cdiv(Ctc, _RC),)
    return pl.pallas_call(
        _tc_body,
        grid=grid,
        in_specs=[
            pl.BlockSpec((_RC, B), lambda i: (i, 0)),
            pl.BlockSpec((1, B), lambda i: (0, 0)),
            pl.BlockSpec((1, B), lambda i: (0, 0)),
        ],
        out_specs=pl.BlockSpec((_RC, B), lambda i: (i, 0)),
        out_shape=jax.ShapeDtypeStruct((Ctc, B), jnp.float32),
    )(xt_top, tgt, pen)


def _t_body(tgt_ref, o_ref):
    o_ref[...] = jnp.full((16,), (1.0 - _MOMENTUM) * jnp.mean(tgt_ref[...]),
                          jnp.float32)


def _t_part(tgt):
    return pl.pallas_call(
        _t_body,
        out_shape=jax.ShapeDtypeStruct((16,), jnp.float32),
    )(tgt)


def _sc_body(x_hbm, t_hbm, pen_hbm, o_hbm, tv, penv, ibuf, obuf,
             insem, outsem):
    C, B = x_hbm.shape
    rows_per_w = C // _NW
    nchunks = rows_per_w // _CH
    cid = lax.axis_index("c")
    sid = lax.axis_index("s")
    wid = sid * _NC + cid
    base = wid * rows_per_w

    pltpu.sync_copy(t_hbm, tv)
    pltpu.sync_copy(pen_hbm, penv)
    t = tv[...]

    def in_copy(i, slot):
        return pltpu.make_async_copy(
            x_hbm.at[pl.ds(base + i * _CH, _CH), :], ibuf.at[slot], insem.at[slot]
        )

    def out_copy(i, slot):
        return pltpu.make_async_copy(
            obuf.at[slot], o_hbm.at[pl.ds(base + i * _CH, _CH), :], outsem.at[slot]
        )

    def compute(slot):
        def row(r, _):
            for j in range(B // 16):
                x = ibuf[slot, r, pl.ds(j * 16, 16)]
                p = penv[pl.ds(j * 16, 16)]
                obuf[slot, r, pl.ds(j * 16, 16)] = jnp.where(x > p, x * (t + x), x)
            return 0

        lax.fori_loop(0, _CH, row, 0)

    for b in range(_NBUF):
        in_copy(b, b).start()

    def group(g, _):
        for b in range(_NBUF):
            i = g * _NBUF + b
            in_copy(i, b).wait()

            @pl.when(i >= _NBUF)
            def _():
                out_copy(i - _NBUF, b).wait()

            compute(b)
            out_copy(i, b).start()

            @pl.when(i + _NBUF < nchunks)
            def _():
                in_copy(i + _NBUF, b).start()

        return 0

    lax.fori_loop(0, nchunks // _NBUF, group, 0)

    for b in range(_NBUF):
        out_copy(nchunks - _NBUF + b, b).wait()


def _sc_part(xt_bot, tvec, pen1):
    Csc, B = xt_bot.shape
    mesh = plsc.VectorSubcoreMesh(core_axis_name="c", subcore_axis_name="s")
    k = functools.partial(
        pl.kernel,
        mesh=mesh,
        out_type=jax.ShapeDtypeStruct((Csc, B), jnp.float32),
        scratch_types=[
            pltpu.VMEM((16,), jnp.float32),
            pltpu.VMEM((B,), jnp.float32),
            pltpu.VMEM((_NBUF, _CH, B), jnp.float32),
            pltpu.VMEM((_NBUF, _CH, B), jnp.float32),
            pltpu.SemaphoreType.DMA((_NBUF,)),
            pltpu.SemaphoreType.DMA((_NBUF,)),
        ],
    )(_sc_body)
    return k(xt_bot, tvec, pen1)


def kernel(cosine_theta, cosine_theta_target, penalty_cosine_theta):
    B, C = cosine_theta.shape
    xt = cosine_theta.T
    tgt = cosine_theta_target.T
    pen = penalty_cosine_theta.T
    ctc = C - _CSC
    tvec = _t_part(tgt)
    tc_out = _tc_part(xt[:ctc], tgt, pen)
    sc_out = _sc_part(xt[ctc:], tvec, penalty_cosine_theta.reshape(B))
    return jnp.concatenate([tc_out, sc_out], axis=0).T


# speedup vs baseline: 7.1473x; 7.1473x over previous
import jax
import jax.numpy as jnp
from jax.experimental import pallas as pl
from jax.experimental.pallas import tpu as pltpu

_MOMENTUM = 0.01
_K = 4

_SIZES = [256, 512, 1024] + [1792] * 53 + [1440] + [1024, 512, 256]
_OFFS = [sum(_SIZES[:i]) for i in range(len(_SIZES))]
_MAXC = max(_SIZES)


def _body(x_hbm, tgt_ref, pen_ref, o_hbm, xbuf, obuf, insem, outsem):
    n = len(_SIZES)
    t = (1.0 - _MOMENTUM) * jnp.mean(tgt_ref[...])
    p = pen_ref[...]

    def in_copy(i, slot):
        return pltpu.make_async_copy(
            x_hbm.at[pl.ds(_OFFS[i], _SIZES[i]), :],
            xbuf.at[slot, pl.ds(0, _SIZES[i]), :],
            insem.at[slot],
        )

    def out_copy(i, slot):
        return pltpu.make_async_copy(
            obuf.at[slot, pl.ds(0, _SIZES[i]), :],
            o_hbm.at[pl.ds(_OFFS[i], _SIZES[i]), :],
            outsem.at[slot],
        )

    for i in range(_K):
        in_copy(i, i).start()

    for i in range(n):
        slot = i % _K
        in_copy(i, slot).wait()
        if i >= _K:
            out_copy(i - _K, slot).wait()
        x = xbuf[slot, pl.ds(0, _SIZES[i]), :]
        obuf[slot, pl.ds(0, _SIZES[i]), :] = jnp.where(x > p, x * (t + x), x)
        out_copy(i, slot).start()
        if i + _K < n:
            in_copy(i + _K, slot).start()

    for i in range(n - _K, n):
        out_copy(i, i % _K).wait()


def kernel(cosine_theta, cosine_theta_target, penalty_cosine_theta):
    B, C = cosine_theta.shape
    xt = cosine_theta.T
    tgt = cosine_theta_target.T
    pen = penalty_cosine_theta.T
    out_t = pl.pallas_call(
        _body,
        in_specs=[
            pl.BlockSpec(memory_space=pl.ANY),
            pl.BlockSpec(memory_space=pltpu.VMEM),
            pl.BlockSpec(memory_space=pltpu.VMEM),
        ],
        out_specs=pl.BlockSpec(memory_space=pl.ANY),
        out_shape=jax.ShapeDtypeStruct((C, B), cosine_theta.dtype),
        scratch_shapes=[
            pltpu.VMEM((_K, _MAXC, B), jnp.float32),
            pltpu.VMEM((_K, _MAXC, B), jnp.float32),
            pltpu.SemaphoreType.DMA((_K,)),
            pltpu.SemaphoreType.DMA((_K,)),
        ],
    )(xt, tgt, pen)
    return out_t.T
